# baseline (speedup 1.0000x reference)
import functools

import jax
import jax.numpy as jnp
from jax import lax
from jax.experimental import pallas as pl
from jax.experimental.pallas import tpu as pltpu
from jax.experimental.pallas import tpu_sc as plsc

_NEMB = 32
_NCHAMP = 1000000
_BATCH = 16384
_NROW4 = _NCHAMP // 4
_NC = 2
_NS = 16
_LANES = 16
_NW = _NC * _NS
_BPW = _BATCH // _NW
_CHUNK = 128
_HALF = _BPW // 2
_BLK = 128
_NFULL = _NCHAMP // _BLK
_TAIL = _NCHAMP - _NFULL * _BLK

_params = pltpu.CompilerParams(
    needs_layout_passes=False, use_tc_tiling_on_sc=True)
_mesh = plsc.VectorSubcoreMesh(core_axis_name="c", subcore_axis_name="s")


@functools.partial(
    pl.kernel,
    out_type=jax.ShapeDtypeStruct((_NROW4, _BLK), jnp.float32),
    mesh=_mesh,
    compiler_params=_params,
    scratch_types=[
        pltpu.VMEM((_NEMB, _BLK), jnp.float32),
        pltpu.VMEM((_NEMB, _BLK), jnp.float32),
        pltpu.VMEM((_NEMB, _BLK), jnp.float32),
        pltpu.VMEM((_NEMB, _BLK), jnp.float32),
        pltpu.VMEM((_TAIL, _NEMB), jnp.float32),
        pltpu.SemaphoreType.DMA,
        pltpu.SemaphoreType.DMA,
    ],
)
def _sc_pack(wt_hbm, tail_hbm, w4_hbm, in_a, in_b, out_a, out_b, tail_v,
             sem_in, sem_out):
    wid = lax.axis_index("s") * _NC + lax.axis_index("c")
    n_i = 244 + jnp.where(wid < _NFULL - 244 * _NW, 1, 0)
    ins = (in_a, in_b)
    outs = (out_a, out_b)
    iota16 = lax.iota(jnp.int32, _LANES)

    def transpose_block(src, dst, nrow):
        for v in range(_BLK // _LANES):
            rows = iota16 + 16 * (v & 1)
            for r in range(nrow):
                col = jnp.full((_LANES,), r * 4 + (v >> 1), jnp.int32)
                dst[r, pl.ds(v * _LANES, _LANES)] = (
                    plsc.load_gather(src, [rows, col]))
        return None

    def start_in(i, buf):
        blk = wid + i * _NW
        return pltpu.async_copy(
            wt_hbm.at[:, pl.ds(blk * _BLK, _BLK)], buf, sem_in)

    def write_out(i, buf):
        blk = wid + i * _NW
        return pltpu.async_copy(
            buf, w4_hbm.at[pl.ds(blk * (_BLK // 4), _BLK // 4), :], sem_out)

    def wait_in(buf):
        pltpu.make_async_copy(
            wt_hbm.at[:, pl.ds(0, _BLK)], buf, sem_in).wait()

    def wait_out(buf):
        pltpu.make_async_copy(
            buf, w4_hbm.at[pl.ds(0, _BLK // 4), :], sem_out).wait()

    start_in(0, ins[0])
    start_in(1, ins[1])

    def body(i2, carry):
        for b in range(2):
            i = i2 * 2 + b

            @pl.when(i < n_i)
            def _(i=i, b=b):
                wait_in(ins[b])

                @pl.when(i >= 2)
                def _():
                    wait_out(outs[b])

                transpose_block(ins[b], outs[b], _NEMB)
                write_out(i, outs[b])

                @pl.when(i + 2 < n_i)
                def _():
                    start_in(i + 2, ins[b])
        return carry

    lax.fori_loop(0, 123, body, 0)
    wait_out(outs[0])
    wait_out(outs[1])

    @pl.when(wid == 0)
    def _():
        pltpu.sync_copy(tail_hbm, tail_v)
        for r in range(_TAIL // 4):
            for v in range(_BLK // _LANES):
                rows = jnp.full((_LANES,), r * 4 + (v >> 1), jnp.int32)
                cols = iota16 + 16 * (v & 1)
                out_a[r, pl.ds(v * _LANES, _LANES)] = (
                    plsc.load_gather(tail_v, [rows, cols]))
        pltpu.async_copy(
            out_a.at[pl.ds(0, _TAIL // 4), :],
            w4_hbm.at[pl.ds(_NFULL * (_BLK // 4), _TAIL // 4), :],
            sem_out).wait()


@functools.partial(
    pl.kernel,
    out_type=jax.ShapeDtypeStruct((_BATCH,), jnp.float32),
    mesh=_mesh,
    compiler_params=_params,
    scratch_types=[
        pltpu.VMEM((_BPW,), jnp.int32),
        pltpu.VMEM((_BPW,), jnp.int32),
        pltpu.VMEM((_BPW,), jnp.int32),
        pltpu.VMEM((_BPW,), jnp.int32),
        pltpu.VMEM((_HALF, _BLK), jnp.float32),
        pltpu.VMEM((_HALF, _BLK), jnp.float32),
        pltpu.VMEM((_BPW,), jnp.float32),
        pltpu.SemaphoreType.DMA,
    ],
)
def _sc_embed_dot(champ1_hbm, champ2_hbm, w4_hbm, out_hbm,
                  idx1_v, idx2_v, grp1_v, grp2_v, rows1_v, rows2_v, out_v,
                  sem):
    wid = lax.axis_index("s") * _NC + lax.axis_index("c")
    base = wid * _BPW
    pltpu.sync_copy(champ1_hbm.at[pl.ds(base, _BPW)], idx1_v)
    pltpu.sync_copy(champ2_hbm.at[pl.ds(base, _BPW)], idx2_v)

    for v in range(_BPW // _LANES):
        sl = pl.ds(v * _LANES, _LANES)
        grp1_v[sl] = lax.shift_right_logical(idx1_v[sl], 2)
        grp2_v[sl] = lax.shift_right_logical(idx2_v[sl], 2)

    for h in range(_BPW // _HALF):
        h0 = h * _HALF
        copies = []
        for j in range(_HALF // _CHUNK):
            src = pl.ds(h0 + j * _CHUNK, _CHUNK)
            dst = pl.ds(j * _CHUNK, _CHUNK)
            copies.append(
                pltpu.async_copy(w4_hbm.at[grp1_v.at[src]], rows1_v.at[dst],
                                 sem))
            copies.append(
                pltpu.async_copy(w4_hbm.at[grp2_v.at[src]], rows2_v.at[dst],
                                 sem))
        for c in copies:
            c.wait()

        def group_body(g, carry):
            local0 = pl.multiple_of(g * _LANES, _LANES)
            sl = pl.ds(h0 + local0, _LANES)
            rows = local0 + lax.iota(jnp.int32, _LANES)
            off1 = lax.shift_left(jnp.bitwise_and(idx1_v[sl], 3), 5)
            off2 = lax.shift_left(jnp.bitwise_and(idx2_v[sl], 3), 5)
            acc = jnp.zeros((_LANES,), jnp.float32)
            for d in range(_NEMB):
                a = plsc.load_gather(rows1_v, [rows, off1 + d])
                b = plsc.load_gather(rows2_v, [rows, off2 + d])
                acc = acc + a * b
            out_v[pl.ds(h0 + local0, _LANES)] = acc
            return carry

        lax.fori_loop(0, _HALF // _LANES, group_body, 0)

    pltpu.sync_copy(out_v, out_hbm.at[pl.ds(base, _BPW)])


def kernel(champ1, champ2, W):
    w4 = _sc_pack(W.T, lax.slice(W, (_NFULL * _BLK, 0), (_NCHAMP, _NEMB)))
    out = _sc_embed_dot(champ1.astype(jnp.int32), champ2.astype(jnp.int32),
                        w4)
    return out.reshape(_BATCH, 1, 1)


# speedup vs baseline: 1.5395x; 1.5395x over previous
import functools

import jax
import jax.numpy as jnp
from jax import lax
from jax.experimental import pallas as pl
from jax.experimental.pallas import tpu as pltpu
from jax.experimental.pallas import tpu_sc as plsc

_NEMB = 32
_BATCH = 16384
_NC = 2
_NS = 16
_LANES = 16
_NW = _NC * _NS
_BPW = _BATCH // _NW
_CHUNK = 128
_NCHUNK = _BPW // _CHUNK


@functools.partial(
    pl.kernel,
    out_type=jax.ShapeDtypeStruct((_BATCH,), jnp.float32),
    mesh=plsc.VectorSubcoreMesh(core_axis_name="c", subcore_axis_name="s"),
    compiler_params=pltpu.CompilerParams(
        needs_layout_passes=False, use_tc_tiling_on_sc=False),
    scratch_types=[
        pltpu.VMEM((_BPW,), jnp.int32),
        pltpu.VMEM((_BPW,), jnp.int32),
        pltpu.VMEM((_BPW, _NEMB), jnp.float32),
        pltpu.VMEM((_BPW, _NEMB), jnp.float32),
        pltpu.VMEM((_BPW,), jnp.float32),
        pltpu.SemaphoreType.DMA,
    ],
)
def _sc_embed_dot(champ1_hbm, champ2_hbm, w_hbm, out_hbm,
                  idx1_v, idx2_v, rows1_v, rows2_v, out_v, sem):
    wid = lax.axis_index("s") * _NC + lax.axis_index("c")
    base = wid * _BPW
    pltpu.sync_copy(champ1_hbm.at[pl.ds(base, _BPW)], idx1_v)
    pltpu.sync_copy(champ2_hbm.at[pl.ds(base, _BPW)], idx2_v)

    copies = []
    for j in range(_NCHUNK):
        sl = pl.ds(j * _CHUNK, _CHUNK)
        copies.append(
            pltpu.async_copy(w_hbm.at[idx1_v.at[sl]], rows1_v.at[sl], sem))
        copies.append(
            pltpu.async_copy(w_hbm.at[idx2_v.at[sl]], rows2_v.at[sl], sem))
    for c in copies:
        c.wait()

    def group_body(g, carry):
        row0 = pl.multiple_of(g * _LANES, _LANES)
        rows = row0 + lax.iota(jnp.int32, _LANES)
        acc = jnp.zeros((_LANES,), jnp.float32)
        for d in range(_NEMB):
            col = jnp.full((_LANES,), d, jnp.int32)
            a = plsc.load_gather(rows1_v, [rows, col])
            b = plsc.load_gather(rows2_v, [rows, col])
            acc = acc + a * b
        out_v[pl.ds(row0, _LANES)] = acc
        return carry

    lax.fori_loop(0, _BPW // _LANES, group_body, 0)
    pltpu.sync_copy(out_v, out_hbm.at[pl.ds(wid * _BPW, _BPW)])


def kernel(champ1, champ2, W):
    out = _sc_embed_dot(champ1.astype(jnp.int32), champ2.astype(jnp.int32), W)
    return out.reshape(_BATCH, 1, 1)


# speedup vs baseline: 4.4425x; 2.8857x over previous
import functools

import jax
import jax.numpy as jnp
from jax import lax
from jax.experimental import pallas as pl
from jax.experimental.pallas import tpu as pltpu
from jax.experimental.pallas import tpu_sc as plsc

_NEMB = 32
_NCHAMP = 1000000
_BATCH = 16384
_NROW4 = _NCHAMP // 4
_NC = 2
_NS = 16
_LANES = 16
_NW = _NC * _NS
_BPW = _BATCH // _NW
_CHUNK = 128
_HALF = _BPW // 2
_BLK = 128
_NFULL = _NCHAMP // _BLK
_TAIL = _NCHAMP - _NFULL * _BLK

_params = pltpu.CompilerParams(
    needs_layout_passes=False, use_tc_tiling_on_sc=True)
_mesh = plsc.VectorSubcoreMesh(core_axis_name="c", subcore_axis_name="s")


@functools.partial(
    pl.kernel,
    out_type=jax.ShapeDtypeStruct((_NROW4, _BLK), jnp.float32),
    mesh=_mesh,
    compiler_params=_params,
    scratch_types=[
        pltpu.VMEM((_NEMB, _BLK), jnp.float32),
        pltpu.VMEM((_NEMB, _BLK), jnp.float32),
        pltpu.VMEM((_NEMB, _BLK), jnp.float32),
        pltpu.VMEM((_NEMB, _BLK), jnp.float32),
        pltpu.VMEM((_TAIL, _NEMB), jnp.float32),
        pltpu.SemaphoreType.DMA,
        pltpu.SemaphoreType.DMA,
    ],
)
def _sc_pack(wt_hbm, tail_hbm, w4_hbm, in_a, in_b, out_a, out_b, tail_v,
             sem_in, sem_out):
    wid = lax.axis_index("s") * _NC + lax.axis_index("c")
    n_i = 244 + jnp.where(wid < _NFULL - 244 * _NW, 1, 0)
    ins = (in_a, in_b)
    outs = (out_a, out_b)
    iota16 = lax.iota(jnp.int32, _LANES)

    def transpose_block(src, dst):
        @functools.partial(plsc.parallel_loop, 0, _BLK // _LANES, unroll=2)
        def _(v):
            rows = iota16 + 16 * jnp.bitwise_and(v, 1)
            vhigh = lax.shift_right_logical(v, 1)
            start = pl.multiple_of(v * _LANES, _LANES)
            for r in range(_NEMB):
                col = jnp.broadcast_to(r * 4 + vhigh, (_LANES,))
                dst[r, pl.ds(start, _LANES)] = (
                    plsc.load_gather(src, [rows, col]))

    def start_in(i, buf):
        blk = wid + i * _NW
        return pltpu.async_copy(
            wt_hbm.at[:, pl.ds(blk * _BLK, _BLK)], buf, sem_in)

    def write_out(i, buf):
        blk = wid + i * _NW
        return pltpu.async_copy(
            buf, w4_hbm.at[pl.ds(blk * (_BLK // 4), _BLK // 4), :], sem_out)

    def wait_in(buf):
        pltpu.make_async_copy(
            wt_hbm.at[:, pl.ds(0, _BLK)], buf, sem_in).wait()

    def wait_out(buf):
        pltpu.make_async_copy(
            buf, w4_hbm.at[pl.ds(0, _BLK // 4), :], sem_out).wait()

    start_in(0, ins[0])
    start_in(1, ins[1])

    def body(i2, carry):
        for b in range(2):
            i = i2 * 2 + b

            @pl.when(i < n_i)
            def _(i=i, b=b):
                wait_in(ins[b])

                @pl.when(i >= 2)
                def _():
                    wait_out(outs[b])

                transpose_block(ins[b], outs[b])
                write_out(i, outs[b])

                @pl.when(i + 2 < n_i)
                def _():
                    start_in(i + 2, ins[b])
        return carry

    lax.fori_loop(0, 123, body, 0)
    wait_out(outs[0])
    wait_out(outs[1])

    @pl.when(wid == 0)
    def _():
        pltpu.sync_copy(tail_hbm, tail_v)
        for r in range(_TAIL // 4):
            for v in range(_BLK // _LANES):
                rows = jnp.full((_LANES,), r * 4 + (v >> 1), jnp.int32)
                cols = iota16 + 16 * (v & 1)
                out_a[r, pl.ds(v * _LANES, _LANES)] = (
                    plsc.load_gather(tail_v, [rows, cols]))
        pltpu.async_copy(
            out_a.at[pl.ds(0, _TAIL // 4), :],
            w4_hbm.at[pl.ds(_NFULL * (_BLK // 4), _TAIL // 4), :],
            sem_out).wait()


@functools.partial(
    pl.kernel,
    out_type=jax.ShapeDtypeStruct((_BATCH,), jnp.float32),
    mesh=_mesh,
    compiler_params=_params,
    scratch_types=[
        pltpu.VMEM((_BPW,), jnp.int32),
        pltpu.VMEM((_BPW,), jnp.int32),
        pltpu.VMEM((_BPW,), jnp.int32),
        pltpu.VMEM((_BPW,), jnp.int32),
        pltpu.VMEM((_HALF, _BLK), jnp.float32),
        pltpu.VMEM((_HALF, _BLK), jnp.float32),
        pltpu.VMEM((_BPW,), jnp.float32),
        pltpu.SemaphoreType.DMA,
    ],
)
def _sc_embed_dot(champ1_hbm, champ2_hbm, w4_hbm, out_hbm,
                  idx1_v, idx2_v, grp1_v, grp2_v, rows1_v, rows2_v, out_v,
                  sem):
    wid = lax.axis_index("s") * _NC + lax.axis_index("c")
    base = wid * _BPW
    pltpu.sync_copy(champ1_hbm.at[pl.ds(base, _BPW)], idx1_v)
    pltpu.sync_copy(champ2_hbm.at[pl.ds(base, _BPW)], idx2_v)

    for v in range(_BPW // _LANES):
        sl = pl.ds(v * _LANES, _LANES)
        grp1_v[sl] = lax.shift_right_logical(idx1_v[sl], 2)
        grp2_v[sl] = lax.shift_right_logical(idx2_v[sl], 2)

    for h in range(_BPW // _HALF):
        h0 = h * _HALF
        copies = []
        for j in range(_HALF // _CHUNK):
            src = pl.ds(h0 + j * _CHUNK, _CHUNK)
            dst = pl.ds(j * _CHUNK, _CHUNK)
            copies.append(
                pltpu.async_copy(w4_hbm.at[grp1_v.at[src]], rows1_v.at[dst],
                                 sem))
            copies.append(
                pltpu.async_copy(w4_hbm.at[grp2_v.at[src]], rows2_v.at[dst],
                                 sem))
        for c in copies:
            c.wait()

        def group_body(g, carry):
            local0 = pl.multiple_of(g * _LANES, _LANES)
            sl = pl.ds(h0 + local0, _LANES)
            rows = local0 + lax.iota(jnp.int32, _LANES)
            off1 = lax.shift_left(jnp.bitwise_and(idx1_v[sl], 3), 5)
            off2 = lax.shift_left(jnp.bitwise_and(idx2_v[sl], 3), 5)
            acc = jnp.zeros((_LANES,), jnp.float32)
            for d in range(_NEMB):
                a = plsc.load_gather(rows1_v, [rows, off1 + d])
                b = plsc.load_gather(rows2_v, [rows, off2 + d])
                acc = acc + a * b
            out_v[pl.ds(h0 + local0, _LANES)] = acc
            return carry

        lax.fori_loop(0, _HALF // _LANES, group_body, 0)

    pltpu.sync_copy(out_v, out_hbm.at[pl.ds(base, _BPW)])


def kernel(champ1, champ2, W):
    w4 = _sc_pack(W.T, lax.slice(W, (_NFULL * _BLK, 0), (_NCHAMP, _NEMB)))
    out = _sc_embed_dot(champ1.astype(jnp.int32), champ2.astype(jnp.int32),
                        w4)
    return out.reshape(_BATCH, 1, 1)


# speedup vs baseline: 4.6019x; 1.0359x over previous
import functools

import jax
import jax.numpy as jnp
from jax import lax
from jax.experimental import pallas as pl
from jax.experimental.pallas import tpu as pltpu
from jax.experimental.pallas import tpu_sc as plsc

_NEMB = 32
_NCHAMP = 1000000
_BATCH = 16384
_NROW4 = _NCHAMP // 4
_NC = 2
_NS = 16
_LANES = 16
_NW = _NC * _NS
_BPW = _BATCH // _NW
_CHUNK = 128
_HALF = _BPW // 2
_BLK = 128
_NFULL = _NCHAMP // _BLK
_TAIL = _NCHAMP - _NFULL * _BLK

_params = pltpu.CompilerParams(
    needs_layout_passes=False, use_tc_tiling_on_sc=True)
_mesh = plsc.VectorSubcoreMesh(core_axis_name="c", subcore_axis_name="s")


@functools.partial(
    pl.kernel,
    out_type=jax.ShapeDtypeStruct((_NROW4, _BLK), jnp.float32),
    mesh=_mesh,
    compiler_params=_params,
    scratch_types=[
        pltpu.VMEM((_NEMB, _BLK), jnp.float32),
        pltpu.VMEM((_NEMB, _BLK), jnp.float32),
        pltpu.VMEM((_NEMB, _BLK), jnp.float32),
        pltpu.VMEM((_NEMB, _BLK), jnp.float32),
        pltpu.VMEM((_TAIL, _NEMB), jnp.float32),
        pltpu.SemaphoreType.DMA,
        pltpu.SemaphoreType.DMA,
    ],
)
def _sc_pack(wt_hbm, tail_hbm, w4_hbm, in_a, in_b, out_a, out_b, tail_v,
             sem_in, sem_out):
    wid = lax.axis_index("s") * _NC + lax.axis_index("c")
    n_i = 244 + jnp.where(wid < _NFULL - 244 * _NW, 1, 0)
    ins = (in_a, in_b)
    outs = (out_a, out_b)
    iota16 = lax.iota(jnp.int32, _LANES)

    def transpose_block(src, dst):
        @functools.partial(plsc.parallel_loop, 0, _BLK // _LANES, unroll=2)
        def _(v):
            rows = iota16 + 16 * jnp.bitwise_and(v, 1)
            vhigh = lax.shift_right_logical(v, 1)
            start = pl.multiple_of(v * _LANES, _LANES)
            for r in range(_NEMB):
                col = jnp.broadcast_to(r * 4 + vhigh, (_LANES,))
                dst[r, pl.ds(start, _LANES)] = (
                    plsc.load_gather(src, [rows, col]))

    def start_in(i, buf):
        blk = wid + i * _NW
        return pltpu.async_copy(
            wt_hbm.at[:, pl.ds(blk * _BLK, _BLK)], buf, sem_in)

    def write_out(i, buf):
        blk = wid + i * _NW
        return pltpu.async_copy(
            buf, w4_hbm.at[pl.ds(blk * (_BLK // 4), _BLK // 4), :], sem_out)

    def wait_in(buf):
        pltpu.make_async_copy(
            wt_hbm.at[:, pl.ds(0, _BLK)], buf, sem_in).wait()

    def wait_out(buf):
        pltpu.make_async_copy(
            buf, w4_hbm.at[pl.ds(0, _BLK // 4), :], sem_out).wait()

    start_in(0, ins[0])
    start_in(1, ins[1])

    def body(i2, carry):
        for b in range(2):
            i = i2 * 2 + b

            @pl.when(i < n_i)
            def _(i=i, b=b):
                wait_in(ins[b])

                @pl.when(i >= 2)
                def _():
                    wait_out(outs[b])

                transpose_block(ins[b], outs[b])
                write_out(i, outs[b])

                @pl.when(i + 2 < n_i)
                def _():
                    start_in(i + 2, ins[b])
        return carry

    lax.fori_loop(0, 123, body, 0)
    wait_out(outs[0])
    wait_out(outs[1])

    @pl.when(wid == 0)
    def _():
        pltpu.sync_copy(tail_hbm, tail_v)
        for r in range(_TAIL // 4):
            for v in range(_BLK // _LANES):
                rows = jnp.full((_LANES,), r * 4 + (v >> 1), jnp.int32)
                cols = iota16 + 16 * (v & 1)
                out_a[r, pl.ds(v * _LANES, _LANES)] = (
                    plsc.load_gather(tail_v, [rows, cols]))
        pltpu.async_copy(
            out_a.at[pl.ds(0, _TAIL // 4), :],
            w4_hbm.at[pl.ds(_NFULL * (_BLK // 4), _TAIL // 4), :],
            sem_out).wait()


@functools.partial(
    pl.kernel,
    out_type=jax.ShapeDtypeStruct((_BATCH,), jnp.float32),
    mesh=_mesh,
    compiler_params=pltpu.CompilerParams(
        needs_layout_passes=False, use_tc_tiling_on_sc=False),
    scratch_types=[
        pltpu.VMEM((_BPW,), jnp.int32),
        pltpu.VMEM((_BPW,), jnp.int32),
        pltpu.VMEM((_BPW, _NEMB), jnp.float32),
        pltpu.VMEM((_BPW, _NEMB), jnp.float32),
        pltpu.VMEM((_BPW,), jnp.float32),
        pltpu.SemaphoreType.DMA,
    ],
)
def _sc_embed_dot(champ1_hbm, champ2_hbm, w_hbm, out_hbm,
                  idx1_v, idx2_v, rows1_v, rows2_v, out_v, sem):
    wid = lax.axis_index("s") * _NC + lax.axis_index("c")
    base = wid * _BPW
    pltpu.sync_copy(champ1_hbm.at[pl.ds(base, _BPW)], idx1_v)
    pltpu.sync_copy(champ2_hbm.at[pl.ds(base, _BPW)], idx2_v)

    copies = []
    for j in range(_BPW // _CHUNK):
        sl = pl.ds(j * _CHUNK, _CHUNK)
        copies.append(
            pltpu.async_copy(w_hbm.at[idx1_v.at[sl]], rows1_v.at[sl], sem))
        copies.append(
            pltpu.async_copy(w_hbm.at[idx2_v.at[sl]], rows2_v.at[sl], sem))
    for c in copies:
        c.wait()

    def group_body(g, carry):
        row0 = pl.multiple_of(g * _LANES, _LANES)
        rows = row0 + lax.iota(jnp.int32, _LANES)
        acc = jnp.zeros((_LANES,), jnp.float32)
        for d in range(_NEMB):
            col = jnp.full((_LANES,), d, jnp.int32)
            a = plsc.load_gather(rows1_v, [rows, col])
            b = plsc.load_gather(rows2_v, [rows, col])
            acc = acc + a * b
        out_v[pl.ds(row0, _LANES)] = acc
        return carry

    lax.fori_loop(0, _BPW // _LANES, group_body, 0)
    pltpu.sync_copy(out_v, out_hbm.at[pl.ds(base, _BPW)])


def kernel(champ1, champ2, W):
    w4 = _sc_pack(W.T, lax.slice(W, (_NFULL * _BLK, 0), (_NCHAMP, _NEMB)))
    w_lin = w4.reshape(_NCHAMP, _NEMB)
    out = _sc_embed_dot(champ1.astype(jnp.int32), champ2.astype(jnp.int32),
                        w_lin)
    return out.reshape(_BATCH, 1, 1)
